# Initial kernel scaffold; baseline (speedup 1.0000x reference)
import jax
import jax.numpy as jnp
from jax.experimental import pallas as pl


def kernel(lctx_words, rctx_words, lctx_entities, rctx_entities, word_table, entity_table):
    raise NotImplementedError("write your pallas kernel here")


# speedup vs baseline: 2.6232x; 2.6232x over previous
import functools

import jax
import jax.numpy as jnp
from jax import lax
from jax.experimental import pallas as pl
from jax.experimental.pallas import tpu as pltpu
from jax.experimental.pallas import tpu_sc as plsc

WE_DIM = 64
EE_DIM = 64
OUT_DIM = WE_DIM + EE_DIM

NC = 2
NS = 16
NW = NC * NS

SUB = 128
CHUNK = 512
NSUB = CHUNK // SUB


def _make_gather(n_rows: int):
    rows_per_w = n_rows // NW
    n_chunks = rows_per_w // CHUNK
    mesh = plsc.VectorSubcoreMesh(core_axis_name="c", subcore_axis_name="s")

    @functools.partial(
        pl.kernel,
        mesh=mesh,
        compiler_params=pltpu.CompilerParams(use_tc_tiling_on_sc=False),
        out_type=jax.ShapeDtypeStruct((n_rows, OUT_DIM), jnp.float32),
        scratch_types=[
            pltpu.VMEM((NSUB, SUB), jnp.int32),
            pltpu.VMEM((NSUB, SUB), jnp.int32),
            pltpu.VMEM((CHUNK, WE_DIM), jnp.float32),
            pltpu.VMEM((CHUNK, EE_DIM), jnp.float32),
            pltpu.SemaphoreType.DMA,
        ],
    )
    def gather_kernel(widx_hbm, eidx_hbm, wtab_hbm, etab_hbm, out_hbm,
                      widx_v, eidx_v, wrows_v, erows_v, sem):
        wid = lax.axis_index("s") * NC + lax.axis_index("c")
        base = wid * rows_per_w
        idx_row0 = wid * (rows_per_w // SUB)

        def body(c, carry):
            cbase = base + c * CHUNK
            crow = idx_row0 + c * NSUB
            pltpu.sync_copy(widx_hbm.at[pl.ds(crow, NSUB)], widx_v)
            pltpu.sync_copy(eidx_hbm.at[pl.ds(crow, NSUB)], eidx_v)
            copies = []
            for j in range(NSUB):
                copies.append(pltpu.async_copy(
                    wtab_hbm.at[widx_v.at[j]],
                    wrows_v.at[pl.ds(j * SUB, SUB)], sem))
                copies.append(pltpu.async_copy(
                    etab_hbm.at[eidx_v.at[j]],
                    erows_v.at[pl.ds(j * SUB, SUB)], sem))
            for cp in copies:
                cp.wait()
            pltpu.sync_copy(wrows_v, out_hbm.at[pl.ds(cbase, CHUNK), pl.ds(0, WE_DIM)])
            pltpu.sync_copy(erows_v, out_hbm.at[pl.ds(cbase, CHUNK), pl.ds(WE_DIM, EE_DIM)])
            return carry

        lax.fori_loop(0, n_chunks, body, 0, unroll=False)

    return gather_kernel


def kernel(lctx_words, rctx_words, lctx_entities, rctx_entities,
           word_table, entity_table):
    b, l = lctx_words.shape
    n_rows = 2 * b * l
    widx = jnp.concatenate(
        [rctx_words.reshape(-1), lctx_words.reshape(-1)]
    ).astype(jnp.int32).reshape(n_rows // SUB, SUB)
    eidx = jnp.concatenate(
        [rctx_entities.reshape(-1), lctx_entities.reshape(-1)]
    ).astype(jnp.int32).reshape(n_rows // SUB, SUB)
    out = _make_gather(n_rows)(widx, eidx, word_table, entity_table)
    return out.reshape(2, b, l, OUT_DIM)


# speedup vs baseline: 2.6302x; 1.0027x over previous
import functools

import jax
import jax.numpy as jnp
from jax import lax
from jax.experimental import pallas as pl
from jax.experimental.pallas import tpu as pltpu
from jax.experimental.pallas import tpu_sc as plsc

WE_DIM = 64
EE_DIM = 64
OUT_DIM = WE_DIM + EE_DIM

NC = 2
NS = 16
NW = NC * NS

SUB = 128
CHUNK = 256
NSUB = CHUNK // SUB


def _make_gather(n_rows: int):
    rows_per_w = n_rows // NW
    n_chunks = rows_per_w // CHUNK
    n_pairs = n_chunks // 2
    assert n_chunks % 2 == 0 and n_pairs >= 2
    mesh = plsc.VectorSubcoreMesh(core_axis_name="c", subcore_axis_name="s")

    @functools.partial(
        pl.kernel,
        mesh=mesh,
        compiler_params=pltpu.CompilerParams(use_tc_tiling_on_sc=False),
        out_type=jax.ShapeDtypeStruct((n_rows, OUT_DIM), jnp.float32),
        scratch_types=[
            pltpu.VMEM((2, NSUB, SUB), jnp.int32),
            pltpu.VMEM((2, NSUB, SUB), jnp.int32),
            pltpu.VMEM((2, CHUNK, WE_DIM), jnp.float32),
            pltpu.VMEM((2, CHUNK, EE_DIM), jnp.float32),
            pltpu.SemaphoreType.DMA,
            pltpu.SemaphoreType.DMA,
            pltpu.SemaphoreType.DMA,
            pltpu.SemaphoreType.DMA,
        ],
    )
    def gather_kernel(widx_hbm, eidx_hbm, wtab_hbm, etab_hbm, out_hbm,
                      widx_v, eidx_v, wrows_v, erows_v,
                      gsem0, gsem1, ssem0, ssem1):
        wid = lax.axis_index("s") * NC + lax.axis_index("c")
        base = wid * rows_per_w
        idx_row0 = wid * (rows_per_w // SUB)
        gsem = (gsem0, gsem1)
        ssem = (ssem0, ssem1)

        def load_idx(c, b):
            crow = idx_row0 + c * NSUB
            pltpu.sync_copy(widx_hbm.at[pl.ds(crow, NSUB)], widx_v.at[b])
            pltpu.sync_copy(eidx_hbm.at[pl.ds(crow, NSUB)], eidx_v.at[b])

        def fire(c, b):
            for j in range(NSUB):
                pltpu.async_copy(wtab_hbm.at[widx_v.at[b, j]],
                                 wrows_v.at[b, pl.ds(j * SUB, SUB)], gsem[b])
                pltpu.async_copy(etab_hbm.at[eidx_v.at[b, j]],
                                 erows_v.at[b, pl.ds(j * SUB, SUB)], gsem[b])

        def _drain(sem, b):
            pltpu.make_async_copy(wtab_hbm.at[pl.ds(0, CHUNK)],
                                  wrows_v.at[b], sem).wait()
            pltpu.make_async_copy(wtab_hbm.at[pl.ds(0, CHUNK)],
                                  erows_v.at[b], sem).wait()

        def wait_g(b):
            _drain(gsem[b], b)

        def store(c, b):
            cbase = base + c * CHUNK
            pltpu.async_copy(wrows_v.at[b],
                             out_hbm.at[pl.ds(cbase, CHUNK), pl.ds(0, WE_DIM)],
                             ssem[b])
            pltpu.async_copy(erows_v.at[b],
                             out_hbm.at[pl.ds(cbase, CHUNK), pl.ds(WE_DIM, EE_DIM)],
                             ssem[b])

        def wait_s(b):
            _drain(ssem[b], b)

        load_idx(0, 0)
        fire(0, 0)
        load_idx(1, 1)
        fire(1, 1)
        wait_g(0)
        store(0, 0)
        wait_s(0)
        load_idx(2, 0)
        fire(2, 0)
        wait_g(1)
        store(1, 1)

        def body(p, carry):
            c0 = 2 * p
            c1 = c0 + 1
            wait_s(1)
            load_idx(c1, 1)
            fire(c1, 1)
            wait_g(0)
            store(c0, 0)
            wait_s(0)
            load_idx(c0 + 2, 0)
            fire(c0 + 2, 0)
            wait_g(1)
            store(c1, 1)
            return carry

        lax.fori_loop(1, n_pairs - 1, body, 0, unroll=False)

        c0 = n_chunks - 2
        wait_s(1)
        load_idx(c0 + 1, 1)
        fire(c0 + 1, 1)
        wait_g(0)
        store(c0, 0)
        wait_g(1)
        store(c0 + 1, 1)
        wait_s(0)
        wait_s(1)

    return gather_kernel


def kernel(lctx_words, rctx_words, lctx_entities, rctx_entities,
           word_table, entity_table):
    b, l = lctx_words.shape
    n_rows = 2 * b * l
    widx = jnp.concatenate(
        [rctx_words.reshape(-1), lctx_words.reshape(-1)]
    ).astype(jnp.int32).reshape(n_rows // SUB, SUB)
    eidx = jnp.concatenate(
        [rctx_entities.reshape(-1), lctx_entities.reshape(-1)]
    ).astype(jnp.int32).reshape(n_rows // SUB, SUB)
    out = _make_gather(n_rows)(widx, eidx, word_table, entity_table)
    return out.reshape(2, b, l, OUT_DIM)
